# baseline (speedup 1.0000x reference)
import jax
import jax.numpy as jnp
from jax import lax
from jax.experimental import pallas as pl
from jax.experimental.pallas import tpu as pltpu


def _chunks(n):
    h = ((n // 2) // 8) * 8
    return [(0, h), (h, n - h)]


def _body(n, d, x_ref, w_ref, xout_ref, score_ref, ob_ref, sems):
    xv = x_ref[...]
    w = w_ref[...]
    w2 = w * lax.rsqrt(jnp.sum(w * w))
    s_row = lax.dot_general(
        w2, xv, (((1,), (1,)), ((), ())), preferred_element_type=jnp.float32
    )
    c = jnp.sum(s_row) / n
    score_ref[...] = jnp.tanh(s_row - c)
    w2t = lax.transpose(w2, (1, 0))
    wb = lax.broadcast_in_dim(w2t, (d, d), (0, 1))
    for i, (off, sz) in enumerate(_chunks(n)):
        xc = lax.slice(xv, (off, 0), (off + sz, d))
        sb = lax.dot_general(
            xc, wb, (((1,), (0,)), ((), ())), preferred_element_type=jnp.float32
        )
        ob_ref[i, pl.ds(0, sz), :] = xc * jnp.tanh(sb - c)
        pltpu.make_async_copy(
            ob_ref.at[i, pl.ds(0, sz), :],
            xout_ref.at[pl.ds(off, sz), :], sems.at[i],
        ).start()
    for i, (off, sz) in enumerate(_chunks(n)):
        pltpu.make_async_copy(
            ob_ref.at[i, pl.ds(0, sz), :],
            xout_ref.at[pl.ds(off, sz), :], sems.at[i],
        ).wait()


def kernel(x, edge_index, weight):
    n, d = x.shape
    max_sz = max(sz for _, sz in _chunks(n))

    def body(*refs):
        _body(n, d, *refs)

    x_out, score = pl.pallas_call(
        body,
        in_specs=[
            pl.BlockSpec((n, d), lambda: (0, 0)),
            pl.BlockSpec((1, d), lambda: (0, 0)),
        ],
        out_specs=[
            pl.BlockSpec(memory_space=pl.ANY),
            pl.BlockSpec((1, n), lambda: (0, 0)),
        ],
        out_shape=(
            jax.ShapeDtypeStruct((n, d), x.dtype),
            jax.ShapeDtypeStruct((1, n), x.dtype),
        ),
        scratch_shapes=[
            pltpu.VMEM((2, max_sz, d), jnp.float32),
            pltpu.SemaphoreType.DMA((2,)),
        ],
    )(x, weight)
    return x_out, score
